# baseline (speedup 1.0000x reference)
import functools

import jax
import jax.numpy as jnp
from jax import lax
from jax.experimental import pallas as pl
from jax.experimental.pallas import tpu as pltpu
from jax.experimental.pallas import tpu_sc as plsc

Q_SIZE = 1024
K_SIZE = 100000
D_SIZE = 64
K_BLK = 4000
NBLK = K_SIZE // K_BLK


def _split3(v):
    u = jax.lax.bitcast_convert_type(v, jnp.uint32)
    hi = jax.lax.bitcast_convert_type(u & jnp.uint32(0xFFFF0000), jnp.float32)
    r1 = v - hi
    u1 = jax.lax.bitcast_convert_type(r1, jnp.uint32)
    mid = jax.lax.bitcast_convert_type(u1 & jnp.uint32(0xFFFF0000), jnp.float32)
    r2 = r1 - mid
    return (hi.astype(jnp.bfloat16), mid.astype(jnp.bfloat16),
            r2.astype(jnp.bfloat16))


def _nn_kernel(xt_ref, mus_ref, min_out_ref, arg_out_ref, xaug_ref):
    pid = pl.program_id(0)

    @pl.when(pid == 0)
    def _prep():
        x_a, x_b, x_c = _split3(xt_ref[...])
        ones = jnp.ones((3, Q_SIZE), jnp.bfloat16)
        xaug_ref[...] = jnp.concatenate([x_a, x_a, x_b, x_c, x_a, x_b, ones],
                                        axis=0)

    mus_blk = mus_ref[...]
    m_a, m_b, m_c = _split3(mus_blk)
    norms = jnp.sum(mus_blk * mus_blk, axis=1, keepdims=True)
    n_a, n_b, n_c = _split3(norms)
    m_aug = jnp.concatenate([m_a, m_b, m_a, m_a, m_c, m_b, n_a, n_b, n_c],
                            axis=1)
    scores = jax.lax.dot_general(
        m_aug, xaug_ref[...], (((1,), (0,)), ((), ())),
        preferred_element_type=jnp.float32,
    )

    min_out_ref[...] = jnp.min(scores, axis=0, keepdims=True)[None]
    blk_arg = jnp.argmin(scores, axis=0).astype(jnp.int32)[None, :]
    arg_out_ref[...] = (blk_arg + pid * K_BLK)[None]


def _sc_merge(mins, args):
    info = plsc.get_sparse_core_info()
    qw = 128
    nw_active = Q_SIZE // qw
    mesh = plsc.VectorSubcoreMesh(core_axis_name="c", subcore_axis_name="s")

    @functools.partial(
        pl.kernel, mesh=mesh,
        out_type=jax.ShapeDtypeStruct((Q_SIZE,), jnp.int32),
        scratch_types=[
            pltpu.VMEM((NBLK, qw), jnp.float32),
            pltpu.VMEM((NBLK, qw), jnp.int32),
            pltpu.VMEM((qw,), jnp.int32),
        ],
    )
    def k(mins_hbm, args_hbm, out_hbm, mins_v, args_v, best_v):
        wid = lax.axis_index("s") * info.num_cores + lax.axis_index("c")

        @pl.when(wid < nw_active)
        def _work():
            base = wid * qw
            pltpu.sync_copy(mins_hbm.at[:, pl.ds(base, qw)], mins_v)
            pltpu.sync_copy(args_hbm.at[:, pl.ds(base, qw)], args_v)
            for h in range(qw // 16):
                sl = pl.ds(h * 16, 16)
                best = mins_v[0, sl]
                bidx = args_v[0, sl]
                for b in range(1, NBLK):
                    vb = mins_v[b, sl]
                    ab = args_v[b, sl]
                    better = vb < best
                    best = jnp.where(better, vb, best)
                    bidx = jnp.where(better, ab, bidx)
                best_v[sl] = bidx
            pltpu.sync_copy(best_v, out_hbm.at[pl.ds(base, qw)])

    return k(mins, args)


def kernel(X, mus):
    xt = -2.0 * X.T

    mins, args = pl.pallas_call(
        _nn_kernel,
        grid=(NBLK,),
        in_specs=[
            pl.BlockSpec((D_SIZE, Q_SIZE), lambda i: (0, 0)),
            pl.BlockSpec((K_BLK, D_SIZE), lambda i: (i, 0)),
        ],
        out_specs=[
            pl.BlockSpec((1, 1, Q_SIZE), lambda i: (i, 0, 0)),
            pl.BlockSpec((1, 1, Q_SIZE), lambda i: (i, 0, 0)),
        ],
        out_shape=[
            jax.ShapeDtypeStruct((NBLK, 1, Q_SIZE), jnp.float32),
            jax.ShapeDtypeStruct((NBLK, 1, Q_SIZE), jnp.int32),
        ],
        scratch_shapes=[
            pltpu.VMEM((6 * D_SIZE + 3, Q_SIZE), jnp.bfloat16),
        ],
    )(xt, mus)
    return _sc_merge(mins.reshape(NBLK, Q_SIZE), args.reshape(NBLK, Q_SIZE))


# speedup vs baseline: 1.0041x; 1.0041x over previous
import functools

import jax
import jax.numpy as jnp
from jax import lax
from jax.experimental import pallas as pl
from jax.experimental.pallas import tpu as pltpu
from jax.experimental.pallas import tpu_sc as plsc

Q_SIZE = 1024
K_SIZE = 100000
D_SIZE = 64
K_BLK = 5000
NBLK = K_SIZE // K_BLK


def _split3(v):
    u = jax.lax.bitcast_convert_type(v, jnp.uint32)
    hi = jax.lax.bitcast_convert_type(u & jnp.uint32(0xFFFF0000), jnp.float32)
    r1 = v - hi
    u1 = jax.lax.bitcast_convert_type(r1, jnp.uint32)
    mid = jax.lax.bitcast_convert_type(u1 & jnp.uint32(0xFFFF0000), jnp.float32)
    r2 = r1 - mid
    return (hi.astype(jnp.bfloat16), mid.astype(jnp.bfloat16),
            r2.astype(jnp.bfloat16))


def _nn_kernel(xt_ref, mus_ref, min_out_ref, arg_out_ref, xaug_ref):
    pid = pl.program_id(0)

    @pl.when(pid == 0)
    def _prep():
        x_a, x_b, x_c = _split3(xt_ref[...])
        ones = jnp.ones((3, Q_SIZE), jnp.bfloat16)
        xaug_ref[...] = jnp.concatenate([x_a, x_a, x_b, x_c, x_a, x_b, ones],
                                        axis=0)

    mus_blk = mus_ref[...]
    m_a, m_b, m_c = _split3(mus_blk)
    norms = jnp.sum(mus_blk * mus_blk, axis=1, keepdims=True)
    n_a, n_b, n_c = _split3(norms)
    m_aug = jnp.concatenate([m_a, m_b, m_a, m_a, m_c, m_b, n_a, n_b, n_c],
                            axis=1)
    scores = jax.lax.dot_general(
        m_aug, xaug_ref[...], (((1,), (0,)), ((), ())),
        preferred_element_type=jnp.float32,
    )

    min_out_ref[...] = jnp.min(scores, axis=0, keepdims=True)[None]
    blk_arg = jnp.argmin(scores, axis=0).astype(jnp.int32)[None, :]
    arg_out_ref[...] = (blk_arg + pid * K_BLK)[None]


def _sc_merge(mins, args):
    info = plsc.get_sparse_core_info()
    qw = 128
    nw_active = Q_SIZE // qw
    mesh = plsc.VectorSubcoreMesh(core_axis_name="c", subcore_axis_name="s")

    @functools.partial(
        pl.kernel, mesh=mesh,
        out_type=jax.ShapeDtypeStruct((Q_SIZE,), jnp.int32),
        scratch_types=[
            pltpu.VMEM((NBLK, qw), jnp.float32),
            pltpu.VMEM((NBLK, qw), jnp.int32),
            pltpu.VMEM((qw,), jnp.int32),
        ],
    )
    def k(mins_hbm, args_hbm, out_hbm, mins_v, args_v, best_v):
        wid = lax.axis_index("s") * info.num_cores + lax.axis_index("c")

        @pl.when(wid < nw_active)
        def _work():
            base = wid * qw
            pltpu.sync_copy(mins_hbm.at[:, pl.ds(base, qw)], mins_v)
            pltpu.sync_copy(args_hbm.at[:, pl.ds(base, qw)], args_v)
            for h in range(qw // 16):
                sl = pl.ds(h * 16, 16)
                best = mins_v[0, sl]
                bidx = args_v[0, sl]
                for b in range(1, NBLK):
                    vb = mins_v[b, sl]
                    ab = args_v[b, sl]
                    better = vb < best
                    best = jnp.where(better, vb, best)
                    bidx = jnp.where(better, ab, bidx)
                best_v[sl] = bidx
            pltpu.sync_copy(best_v, out_hbm.at[pl.ds(base, qw)])

    return k(mins, args)


def kernel(X, mus):
    xt = -2.0 * X.T

    mins, args = pl.pallas_call(
        _nn_kernel,
        grid=(NBLK,),
        in_specs=[
            pl.BlockSpec((D_SIZE, Q_SIZE), lambda i: (0, 0)),
            pl.BlockSpec((K_BLK, D_SIZE), lambda i: (i, 0)),
        ],
        out_specs=[
            pl.BlockSpec((1, 1, Q_SIZE), lambda i: (i, 0, 0)),
            pl.BlockSpec((1, 1, Q_SIZE), lambda i: (i, 0, 0)),
        ],
        out_shape=[
            jax.ShapeDtypeStruct((NBLK, 1, Q_SIZE), jnp.float32),
            jax.ShapeDtypeStruct((NBLK, 1, Q_SIZE), jnp.int32),
        ],
        scratch_shapes=[
            pltpu.VMEM((6 * D_SIZE + 3, Q_SIZE), jnp.bfloat16),
        ],
    )(xt, mus)
    return _sc_merge(mins.reshape(NBLK, Q_SIZE), args.reshape(NBLK, Q_SIZE))
